# baseline (speedup 1.0000x reference)
import functools

import jax
import jax.numpy as jnp
from jax import lax
from jax.experimental import pallas as pl
from jax.experimental.pallas import tpu as pltpu
from jax.experimental.pallas import tpu_sc as plsc

N = 10000
E = 320000
D = 128
OUT = 40

NC = 2
NS = 16
NW = NC * NS
EPW = E // NW
KB = 80
NBATCH = EPW // KB
KBP = 64
NBP = 160
EPWPD = KBP * NBP
NPAD = 10240
RPS = NPAD // NS
RCHUNK = 80
NCHUNK = RPS // RCHUNK
DEGW = 16

BLK = 2000
GRID = N // BLK


def _deg_body(dst_hbm, out_hbm, deg_sh, dst_v, ones_v, zero_v):
    c = lax.axis_index("c")
    s = lax.axis_index("s")
    wid = s * NC + c

    def fill_ones(i, _):
        ones_v[i, :] = jnp.ones((DEGW,), jnp.float32)
        return 0
    lax.fori_loop(0, KB, fill_ones, 0)

    def fill_zero(i, _):
        zero_v[i, :] = jnp.zeros((DEGW,), jnp.float32)
        return 0
    lax.fori_loop(0, RPS, fill_zero, 0)

    rbase = s * RPS
    pltpu.sync_copy(zero_v, deg_sh.at[pl.ds(rbase, RPS)])
    plsc.subcore_barrier()

    ebase = wid * EPW

    def ebody(b, _):
        off = ebase + b * KB
        pltpu.sync_copy(dst_hbm.at[pl.ds(off, KB)], dst_v)
        pltpu.sync_copy(ones_v, deg_sh.at[dst_v], add=True)
        return 0
    lax.fori_loop(0, NBATCH, ebody, 0)
    plsc.subcore_barrier()

    pltpu.sync_copy(deg_sh.at[pl.ds(rbase, RPS)], out_hbm.at[c, pl.ds(rbase, RPS)])


def _sc_deg(dst):
    mesh = plsc.VectorSubcoreMesh(core_axis_name="c", subcore_axis_name="s")
    f = pl.kernel(
        _deg_body,
        out_type=jax.ShapeDtypeStruct((NC, NPAD, DEGW), jnp.float32),
        mesh=mesh,
        scratch_types=[
            pltpu.VMEM_SHARED((NPAD, DEGW), jnp.float32),
            pltpu.VMEM((KB,), jnp.int32),
            pltpu.VMEM((KB, DEGW), jnp.float32),
            pltpu.VMEM((RPS, DEGW), jnp.float32),
        ],
    )
    return f(dst)


def _agg_body(y_hbm, src_hbm, dst_hbm, out_hbm, acc,
              si_a, si_b, di_a, di_b, rows_a, rows_b, gsem, ssem):
    c = lax.axis_index("c")
    s = lax.axis_index("s")
    wid = s * NC + c

    nlane = D // 16

    def fill_zero(k, _):
        i = k // nlane
        j = k % nlane
        rows_a[i, pl.ds(j * 16, 16)] = jnp.zeros((16,), jnp.float32)
        return 0
    lax.fori_loop(0, KBP * nlane, fill_zero, 0)

    rbase = s * RPS
    for jj in range(NCHUNK):
        pltpu.sync_copy(rows_a.at[pl.ds(0, RCHUNK)],
                        acc.at[pl.ds(rbase + jj * RCHUNK, RCHUNK)])
    plsc.subcore_barrier()

    ebase = wid * EPWPD

    def pair(p, _):
        off0 = ebase + (2 * p) * KBP
        off1 = off0 + KBP
        pltpu.sync_copy(src_hbm.at[pl.ds(off0, KBP)], si_a)
        g0 = pltpu.async_copy(y_hbm.at[si_a], rows_a, gsem)
        pltpu.sync_copy(src_hbm.at[pl.ds(off1, KBP)], si_b)
        pltpu.sync_copy(dst_hbm.at[pl.ds(off0, KBP)], di_a)
        g0.wait()
        g1 = pltpu.async_copy(y_hbm.at[si_b], rows_b, gsem)
        s0 = pltpu.async_copy(rows_a, acc.at[di_a], ssem, add=True)
        pltpu.sync_copy(dst_hbm.at[pl.ds(off1, KBP)], di_b)
        g1.wait()
        s0.wait()
        pltpu.sync_copy(rows_b, acc.at[di_b], add=True)
        return 0
    lax.fori_loop(0, NBP // 2, pair, 0)
    plsc.subcore_barrier()

    for jj in range(NCHUNK):
        sl = pl.ds(rbase + jj * RCHUNK, RCHUNK)
        pltpu.sync_copy(acc.at[sl], out_hbm.at[c, sl])


def _sc_agg(y, src, dst):
    mesh = plsc.VectorSubcoreMesh(core_axis_name="c", subcore_axis_name="s")
    f = pl.kernel(
        _agg_body,
        out_type=jax.ShapeDtypeStruct((NC, NPAD, D), jnp.float32),
        mesh=mesh,
        scratch_types=(
            [pltpu.VMEM_SHARED((NPAD, D), jnp.float32)]
            + [pltpu.VMEM((KBP,), jnp.int32)] * 4
            + [pltpu.VMEM((KBP, D), jnp.float32)] * 2
            + [pltpu.SemaphoreType.DMA] * 2
        ),
    )
    return f(y, src, dst)


def _prep_kernel(hist_ref, x_ref, w_ref, y_ref, dinv_ref):
    h = hist_ref[0] + hist_ref[1]
    cnt = jnp.sum(h, axis=1, keepdims=True) * (1.0 / DEGW)
    dinv = lax.rsqrt(1.0 + cnt)
    xw = jnp.dot(x_ref[...], w_ref[...], preferred_element_type=jnp.float32)
    y_ref[...] = dinv * xw
    dinv_ref[...] = jnp.broadcast_to(dinv, (BLK, D))


def _tc_prep(hist, x, w0):
    return pl.pallas_call(
        _prep_kernel,
        grid=(GRID,),
        in_specs=[
            pl.BlockSpec((NC, BLK, DEGW), lambda i: (0, i, 0)),
            pl.BlockSpec((BLK, D), lambda i: (i, 0)),
            pl.BlockSpec((D, D), lambda i: (0, 0)),
        ],
        out_specs=[
            pl.BlockSpec((BLK, D), lambda i: (i, 0)),
            pl.BlockSpec((BLK, D), lambda i: (i, 0)),
        ],
        out_shape=[
            jax.ShapeDtypeStruct((N, D), jnp.float32),
            jax.ShapeDtypeStruct((N, D), jnp.float32),
        ],
    )(hist, x, w0)


def _mid_kernel(s_ref, y_ref, dinv_ref, b_ref, w_ref, o_ref):
    t = dinv_ref[...] * (s_ref[0] + s_ref[1] + y_ref[...]) + b_ref[...]
    h = jnp.maximum(t, 0.0)
    o_ref[...] = dinv_ref[...] * jnp.dot(
        h, w_ref[...], preferred_element_type=jnp.float32)


def _tc_mid(s, y, dinvb, b, w):
    return pl.pallas_call(
        _mid_kernel,
        grid=(GRID,),
        in_specs=[
            pl.BlockSpec((NC, BLK, D), lambda i: (0, i, 0)),
            pl.BlockSpec((BLK, D), lambda i: (i, 0)),
            pl.BlockSpec((BLK, D), lambda i: (i, 0)),
            pl.BlockSpec((1, D), lambda i: (0, 0)),
            pl.BlockSpec((D, D), lambda i: (0, 0)),
        ],
        out_specs=pl.BlockSpec((BLK, D), lambda i: (i, 0)),
        out_shape=jax.ShapeDtypeStruct((N, D), jnp.float32),
    )(s, y, dinvb, b, w)


def _fin_kernel(s_ref, y_ref, dinv_ref, b_ref, wfc_ref, bfc_ref, o_ref):
    t = dinv_ref[...] * (s_ref[0] + s_ref[1] + y_ref[...]) + b_ref[...]
    h = jnp.maximum(t, 0.0)
    logits = jnp.dot(h, wfc_ref[...],
                     preferred_element_type=jnp.float32) + bfc_ref[...]
    m = jnp.max(logits, axis=1, keepdims=True)
    lse = m + jnp.log(jnp.sum(jnp.exp(logits - m), axis=1, keepdims=True))
    o_ref[...] = logits - lse


def _tc_fin(s, y, dinvb, b, wfc_p, bfc_p):
    return pl.pallas_call(
        _fin_kernel,
        grid=(GRID,),
        in_specs=[
            pl.BlockSpec((NC, BLK, D), lambda i: (0, i, 0)),
            pl.BlockSpec((BLK, D), lambda i: (i, 0)),
            pl.BlockSpec((BLK, D), lambda i: (i, 0)),
            pl.BlockSpec((1, D), lambda i: (0, 0)),
            pl.BlockSpec((D, D), lambda i: (0, 0)),
            pl.BlockSpec((1, D), lambda i: (0, 0)),
        ],
        out_specs=pl.BlockSpec((BLK, D), lambda i: (i, 0)),
        out_shape=jax.ShapeDtypeStruct((N, D), jnp.float32),
    )(s, y, dinvb, b, wfc_p, bfc_p)


def kernel(x, edge_index, W0, b0, W1, b1, W2, b2, Wfc, bfc):
    src = edge_index[0]
    dst = edge_index[1]
    srcf = jnp.pad(src.reshape(NW, EPW),
                   ((0, 0), (0, EPWPD - EPW))).reshape(NW * EPWPD)
    pad_dst = N + jnp.arange(EPWPD - EPW, dtype=jnp.int32) % (NPAD - N)
    dstf = jnp.concatenate(
        [dst.reshape(NW, EPW),
         jnp.broadcast_to(pad_dst, (NW, EPWPD - EPW))],
        axis=1).reshape(NW * EPWPD)
    hist = _sc_deg(dst)
    y0, dinvb = _tc_prep(hist, x, W0)

    s0 = _sc_agg(y0, srcf, dstf)
    y1 = _tc_mid(s0, y0, dinvb, b0.reshape(1, D), W1)

    s1 = _sc_agg(y1, srcf, dstf)
    y2 = _tc_mid(s1, y1, dinvb, b1.reshape(1, D), W2)

    s2 = _sc_agg(y2, srcf, dstf)
    wfc_p = jnp.zeros((D, D), jnp.float32).at[:, :OUT].set(Wfc)
    bfc_p = jnp.full((1, D), -1e30, jnp.float32).at[0, :OUT].set(bfc)
    logp = _tc_fin(s2, y2, dinvb, b2.reshape(1, D), wfc_p, bfc_p)
    return logp[:, :OUT]


# speedup vs baseline: 1.0496x; 1.0496x over previous
import functools

import jax
import jax.numpy as jnp
from jax import lax
from jax.experimental import pallas as pl
from jax.experimental.pallas import tpu as pltpu
from jax.experimental.pallas import tpu_sc as plsc

N = 10000
E = 320000
D = 128
OUT = 40

NC = 2
NS = 16
NW = NC * NS
EPW = E // NW
KB = 80
NBATCH = EPW // KB
KBP = 80
NBP = 128
EPWPD = KBP * NBP
NPAD = 10240
RPS = NPAD // NS
RCHUNK = 80
NCHUNK = RPS // RCHUNK
DEGW = 16

BLK = 2000
GRID = N // BLK


def _deg_body(dst_hbm, out_hbm, deg_sh, di_a, di_b, ones_v, zero_v, dsem):
    c = lax.axis_index("c")
    s = lax.axis_index("s")
    wid = s * NC + c

    def fill_ones(i, _):
        ones_v[i, :] = jnp.ones((DEGW,), jnp.float32)
        return 0
    lax.fori_loop(0, KB, fill_ones, 0)

    def fill_zero(i, _):
        zero_v[i, :] = jnp.zeros((DEGW,), jnp.float32)
        return 0
    lax.fori_loop(0, RPS, fill_zero, 0)

    rbase = s * RPS
    pltpu.sync_copy(zero_v, deg_sh.at[pl.ds(rbase, RPS)])
    plsc.subcore_barrier()

    ebase = wid * EPW

    def ebody(p, _):
        off0 = ebase + (2 * p) * KB
        off1 = off0 + KB
        pltpu.sync_copy(dst_hbm.at[pl.ds(off0, KB)], di_a)
        s0 = pltpu.async_copy(ones_v, deg_sh.at[di_a], dsem, add=True)
        pltpu.sync_copy(dst_hbm.at[pl.ds(off1, KB)], di_b)
        s0.wait()
        pltpu.sync_copy(ones_v, deg_sh.at[di_b], add=True)
        return 0
    lax.fori_loop(0, NBATCH // 2, ebody, 0)
    off = ebase + (NBATCH - 1) * KB
    pltpu.sync_copy(dst_hbm.at[pl.ds(off, KB)], di_a)
    pltpu.sync_copy(ones_v, deg_sh.at[di_a], add=True)
    plsc.subcore_barrier()

    pltpu.sync_copy(deg_sh.at[pl.ds(rbase, RPS)], out_hbm.at[c, pl.ds(rbase, RPS)])


def _sc_deg(dst):
    mesh = plsc.VectorSubcoreMesh(core_axis_name="c", subcore_axis_name="s")
    f = pl.kernel(
        _deg_body,
        out_type=jax.ShapeDtypeStruct((NC, NPAD, DEGW), jnp.float32),
        mesh=mesh,
        scratch_types=[
            pltpu.VMEM_SHARED((NPAD, DEGW), jnp.float32),
            pltpu.VMEM((KB,), jnp.int32),
            pltpu.VMEM((KB,), jnp.int32),
            pltpu.VMEM((KB, DEGW), jnp.float32),
            pltpu.VMEM((RPS, DEGW), jnp.float32),
            pltpu.SemaphoreType.DMA,
        ],
    )
    return f(dst)


def _agg_body(y_hbm, src_hbm, dst_hbm, out_hbm, acc,
              si_a, si_b, di_a, di_b, rows_a, rows_b, gsem, ssem):
    c = lax.axis_index("c")
    s = lax.axis_index("s")
    wid = s * NC + c

    nlane = D // 16

    def fill_zero(k, _):
        i = k // nlane
        j = k % nlane
        rows_a[i, pl.ds(j * 16, 16)] = jnp.zeros((16,), jnp.float32)
        return 0
    lax.fori_loop(0, KBP * nlane, fill_zero, 0)

    rbase = s * RPS
    for jj in range(NCHUNK):
        pltpu.sync_copy(rows_a.at[pl.ds(0, RCHUNK)],
                        acc.at[pl.ds(rbase + jj * RCHUNK, RCHUNK)])
    plsc.subcore_barrier()

    ebase = wid * EPWPD

    def pair(p, _):
        off0 = ebase + (2 * p) * KBP
        off1 = off0 + KBP
        pltpu.sync_copy(src_hbm.at[pl.ds(off0, KBP)], si_a)
        g0 = pltpu.async_copy(y_hbm.at[si_a], rows_a, gsem)
        pltpu.sync_copy(src_hbm.at[pl.ds(off1, KBP)], si_b)
        pltpu.sync_copy(dst_hbm.at[pl.ds(off0, KBP)], di_a)
        g0.wait()
        g1 = pltpu.async_copy(y_hbm.at[si_b], rows_b, gsem)
        s0 = pltpu.async_copy(rows_a, acc.at[di_a], ssem, add=True)
        pltpu.sync_copy(dst_hbm.at[pl.ds(off1, KBP)], di_b)
        g1.wait()
        s0.wait()
        pltpu.sync_copy(rows_b, acc.at[di_b], add=True)
        return 0
    lax.fori_loop(0, NBP // 2, pair, 0)
    plsc.subcore_barrier()

    for jj in range(NCHUNK):
        sl = pl.ds(rbase + jj * RCHUNK, RCHUNK)
        pltpu.sync_copy(acc.at[sl], out_hbm.at[c, sl])


def _sc_agg(y, src, dst):
    mesh = plsc.VectorSubcoreMesh(core_axis_name="c", subcore_axis_name="s")
    f = pl.kernel(
        _agg_body,
        out_type=jax.ShapeDtypeStruct((NC, NPAD, D), jnp.float32),
        mesh=mesh,
        scratch_types=(
            [pltpu.VMEM_SHARED((NPAD, D), jnp.float32)]
            + [pltpu.VMEM((KBP,), jnp.int32)] * 4
            + [pltpu.VMEM((KBP, D), jnp.float32)] * 2
            + [pltpu.SemaphoreType.DMA] * 2
        ),
    )
    return f(y, src, dst)


def _prep_kernel(hist_ref, x_ref, w_ref, y_ref, dinv_ref):
    h = hist_ref[0] + hist_ref[1]
    cnt = jnp.sum(h, axis=1, keepdims=True) * (1.0 / DEGW)
    dinv = lax.rsqrt(1.0 + cnt)
    xw = jnp.dot(x_ref[...], w_ref[...], preferred_element_type=jnp.float32)
    y_ref[...] = dinv * xw
    dinv_ref[...] = jnp.broadcast_to(dinv, (BLK, D))


def _tc_prep(hist, x, w0):
    return pl.pallas_call(
        _prep_kernel,
        grid=(GRID,),
        in_specs=[
            pl.BlockSpec((NC, BLK, DEGW), lambda i: (0, i, 0)),
            pl.BlockSpec((BLK, D), lambda i: (i, 0)),
            pl.BlockSpec((D, D), lambda i: (0, 0)),
        ],
        out_specs=[
            pl.BlockSpec((BLK, D), lambda i: (i, 0)),
            pl.BlockSpec((BLK, D), lambda i: (i, 0)),
        ],
        out_shape=[
            jax.ShapeDtypeStruct((N, D), jnp.float32),
            jax.ShapeDtypeStruct((N, D), jnp.float32),
        ],
    )(hist, x, w0)


def _mid_kernel(s_ref, y_ref, dinv_ref, b_ref, w_ref, o_ref):
    t = dinv_ref[...] * (s_ref[0] + s_ref[1] + y_ref[...]) + b_ref[...]
    h = jnp.maximum(t, 0.0)
    o_ref[...] = dinv_ref[...] * jnp.dot(
        h, w_ref[...], preferred_element_type=jnp.float32)


def _tc_mid(s, y, dinvb, b, w):
    return pl.pallas_call(
        _mid_kernel,
        grid=(GRID,),
        in_specs=[
            pl.BlockSpec((NC, BLK, D), lambda i: (0, i, 0)),
            pl.BlockSpec((BLK, D), lambda i: (i, 0)),
            pl.BlockSpec((BLK, D), lambda i: (i, 0)),
            pl.BlockSpec((1, D), lambda i: (0, 0)),
            pl.BlockSpec((D, D), lambda i: (0, 0)),
        ],
        out_specs=pl.BlockSpec((BLK, D), lambda i: (i, 0)),
        out_shape=jax.ShapeDtypeStruct((N, D), jnp.float32),
    )(s, y, dinvb, b, w)


def _fin_kernel(s_ref, y_ref, dinv_ref, b_ref, wfc_ref, bfc_ref, o_ref):
    t = dinv_ref[...] * (s_ref[0] + s_ref[1] + y_ref[...]) + b_ref[...]
    h = jnp.maximum(t, 0.0)
    logits = jnp.dot(h, wfc_ref[...],
                     preferred_element_type=jnp.float32) + bfc_ref[...]
    m = jnp.max(logits, axis=1, keepdims=True)
    lse = m + jnp.log(jnp.sum(jnp.exp(logits - m), axis=1, keepdims=True))
    o_ref[...] = logits - lse


def _tc_fin(s, y, dinvb, b, wfc_p, bfc_p):
    return pl.pallas_call(
        _fin_kernel,
        grid=(GRID,),
        in_specs=[
            pl.BlockSpec((NC, BLK, D), lambda i: (0, i, 0)),
            pl.BlockSpec((BLK, D), lambda i: (i, 0)),
            pl.BlockSpec((BLK, D), lambda i: (i, 0)),
            pl.BlockSpec((1, D), lambda i: (0, 0)),
            pl.BlockSpec((D, D), lambda i: (0, 0)),
            pl.BlockSpec((1, D), lambda i: (0, 0)),
        ],
        out_specs=pl.BlockSpec((BLK, D), lambda i: (i, 0)),
        out_shape=jax.ShapeDtypeStruct((N, D), jnp.float32),
    )(s, y, dinvb, b, wfc_p, bfc_p)


def kernel(x, edge_index, W0, b0, W1, b1, W2, b2, Wfc, bfc):
    src = edge_index[0]
    dst = edge_index[1]
    srcf = jnp.pad(src.reshape(NW, EPW),
                   ((0, 0), (0, EPWPD - EPW))).reshape(NW * EPWPD)
    pad_dst = N + jnp.arange(EPWPD - EPW, dtype=jnp.int32) % (NPAD - N)
    dstf = jnp.concatenate(
        [dst.reshape(NW, EPW),
         jnp.broadcast_to(pad_dst, (NW, EPWPD - EPW))],
        axis=1).reshape(NW * EPWPD)
    hist = _sc_deg(dst)
    y0, dinvb = _tc_prep(hist, x, W0)

    s0 = _sc_agg(y0, srcf, dstf)
    y1 = _tc_mid(s0, y0, dinvb, b0.reshape(1, D), W1)

    s1 = _sc_agg(y1, srcf, dstf)
    y2 = _tc_mid(s1, y1, dinvb, b1.reshape(1, D), W2)

    s2 = _sc_agg(y2, srcf, dstf)
    wfc_p = jnp.zeros((D, D), jnp.float32).at[:, :OUT].set(Wfc)
    bfc_p = jnp.full((1, D), -1e30, jnp.float32).at[0, :OUT].set(bfc)
    logp = _tc_fin(s2, y2, dinvb, b2.reshape(1, D), wfc_p, bfc_p)
    return logp[:, :OUT]


# speedup vs baseline: 2.1966x; 2.0927x over previous
import functools

import jax
import jax.numpy as jnp
from jax import lax
from jax.experimental import pallas as pl
from jax.experimental.pallas import tpu as pltpu
from jax.experimental.pallas import tpu_sc as plsc

N = 10000
E = 320000
D = 128
OUT = 40

NC = 2
NS = 16
NW = NC * NS
EPW = E // NW
KB = 80
NBATCH = EPW // KB
KBP = 80
NBP = 128
EPWPD = KBP * NBP
NPAD = 10240
RPS = NPAD // NS
RCHUNK = 80
NCHUNK = RPS // RCHUNK
DEGW = 16

BLK = 2000
GRID = N // BLK


def _deg_body(dst_hbm, out_hbm, deg_sh, di_a, di_b, ones_v, zero_v, dsem):
    c = lax.axis_index("c")
    s = lax.axis_index("s")
    wid = s * NC + c

    def fill_ones(i, _):
        ones_v[i, :] = jnp.ones((DEGW,), jnp.float32)
        return 0
    lax.fori_loop(0, KB, fill_ones, 0)

    def fill_zero(i, _):
        zero_v[i, :] = jnp.zeros((DEGW,), jnp.float32)
        return 0
    lax.fori_loop(0, RPS, fill_zero, 0)

    rbase = s * RPS
    pltpu.sync_copy(zero_v, deg_sh.at[pl.ds(rbase, RPS)])
    plsc.subcore_barrier()

    ebase = wid * EPW

    def ebody(p, _):
        off0 = ebase + (2 * p) * KB
        off1 = off0 + KB
        pltpu.sync_copy(dst_hbm.at[pl.ds(off0, KB)], di_a)
        s0 = pltpu.async_copy(ones_v, deg_sh.at[di_a], dsem, add=True)
        pltpu.sync_copy(dst_hbm.at[pl.ds(off1, KB)], di_b)
        s0.wait()
        pltpu.sync_copy(ones_v, deg_sh.at[di_b], add=True)
        return 0
    lax.fori_loop(0, NBATCH // 2, ebody, 0)
    off = ebase + (NBATCH - 1) * KB
    pltpu.sync_copy(dst_hbm.at[pl.ds(off, KB)], di_a)
    pltpu.sync_copy(ones_v, deg_sh.at[di_a], add=True)
    plsc.subcore_barrier()

    pltpu.sync_copy(deg_sh.at[pl.ds(rbase, RPS)], out_hbm.at[c, pl.ds(rbase, RPS)])


def _sc_deg(dst):
    mesh = plsc.VectorSubcoreMesh(core_axis_name="c", subcore_axis_name="s")
    f = pl.kernel(
        _deg_body,
        out_type=jax.ShapeDtypeStruct((NC, NPAD, DEGW), jnp.float32),
        mesh=mesh,
        scratch_types=[
            pltpu.VMEM_SHARED((NPAD, DEGW), jnp.float32),
            pltpu.VMEM((KB,), jnp.int32),
            pltpu.VMEM((KB,), jnp.int32),
            pltpu.VMEM((KB, DEGW), jnp.float32),
            pltpu.VMEM((RPS, DEGW), jnp.float32),
            pltpu.SemaphoreType.DMA,
        ],
    )
    return f(dst)


def _agg_body(y_hbm, src_hbm, dst_hbm, out_hbm, acc,
              si_a, si_b, di_a, di_b, rows_a, rows_b, gsem, ssem):
    c = lax.axis_index("c")
    s = lax.axis_index("s")
    wid = s * NC + c

    nlane = D // 16

    def fill_zero(k, _):
        i = k // nlane
        j = k % nlane
        rows_a[i, pl.ds(j * 16, 16)] = jnp.zeros((16,), jnp.float32)
        return 0
    lax.fori_loop(0, KB * nlane, fill_zero, 0)

    rbase = s * RPS
    for jj in range(NCHUNK):
        pltpu.sync_copy(rows_a, acc.at[pl.ds(rbase + jj * RCHUNK, RCHUNK)])
    plsc.subcore_barrier()

    ebase = wid * EPW

    def pair(p, _):
        off0 = ebase + (2 * p) * KB
        off1 = off0 + KB
        pltpu.sync_copy(src_hbm.at[pl.ds(off0, KB)], si_a)
        g0 = pltpu.async_copy(y_hbm.at[si_a], rows_a, gsem)
        pltpu.sync_copy(src_hbm.at[pl.ds(off1, KB)], si_b)
        pltpu.sync_copy(dst_hbm.at[pl.ds(off0, KB)], di_a)
        g0.wait()
        g1 = pltpu.async_copy(y_hbm.at[si_b], rows_b, gsem)
        s0 = pltpu.async_copy(rows_a, acc.at[di_a], ssem, add=True)
        pltpu.sync_copy(dst_hbm.at[pl.ds(off1, KB)], di_b)
        g1.wait()
        s0.wait()
        pltpu.sync_copy(rows_b, acc.at[di_b], add=True)
        return 0
    lax.fori_loop(0, NBATCH // 2, pair, 0)

    off = ebase + (NBATCH - 1) * KB
    pltpu.sync_copy(src_hbm.at[pl.ds(off, KB)], si_a)
    pltpu.async_copy(y_hbm.at[si_a], rows_a, gsem).wait()
    pltpu.sync_copy(dst_hbm.at[pl.ds(off, KB)], di_a)
    pltpu.sync_copy(rows_a, acc.at[di_a], add=True)
    plsc.subcore_barrier()

    for jj in range(NCHUNK):
        sl = pl.ds(rbase + jj * RCHUNK, RCHUNK)
        pltpu.sync_copy(acc.at[sl], out_hbm.at[c, sl])


def _sc_agg(y, src, dst):
    mesh = plsc.VectorSubcoreMesh(core_axis_name="c", subcore_axis_name="s")
    f = pl.kernel(
        _agg_body,
        out_type=jax.ShapeDtypeStruct((NC, NPAD, D), jnp.float32),
        mesh=mesh,
        scratch_types=(
            [pltpu.VMEM_SHARED((NPAD, D), jnp.float32)]
            + [pltpu.VMEM((KB,), jnp.int32)] * 4
            + [pltpu.VMEM((KB, D), jnp.float32)] * 2
            + [pltpu.SemaphoreType.DMA] * 2
        ),
    )
    return f(y, src, dst)


def _prep_kernel(hist_ref, x_ref, w_ref, y_ref, dinv_ref):
    h = hist_ref[0] + hist_ref[1]
    cnt = jnp.sum(h, axis=1, keepdims=True) * (1.0 / DEGW)
    dinv = lax.rsqrt(1.0 + cnt)
    xw = jnp.dot(x_ref[...], w_ref[...], preferred_element_type=jnp.float32)
    y_ref[...] = dinv * xw
    dinv_ref[...] = jnp.broadcast_to(dinv, (BLK, D))


def _tc_prep(hist, x, w0):
    return pl.pallas_call(
        _prep_kernel,
        grid=(GRID,),
        in_specs=[
            pl.BlockSpec((NC, BLK, DEGW), lambda i: (0, i, 0)),
            pl.BlockSpec((BLK, D), lambda i: (i, 0)),
            pl.BlockSpec((D, D), lambda i: (0, 0)),
        ],
        out_specs=[
            pl.BlockSpec((BLK, D), lambda i: (i, 0)),
            pl.BlockSpec((BLK, D), lambda i: (i, 0)),
        ],
        out_shape=[
            jax.ShapeDtypeStruct((N, D), jnp.float32),
            jax.ShapeDtypeStruct((N, D), jnp.float32),
        ],
    )(hist, x, w0)


def _mid_kernel(s_ref, y_ref, dinv_ref, b_ref, w_ref, o_ref):
    t = dinv_ref[...] * (s_ref[0] + s_ref[1] + y_ref[...]) + b_ref[...]
    h = jnp.maximum(t, 0.0)
    o_ref[...] = dinv_ref[...] * jnp.dot(
        h, w_ref[...], preferred_element_type=jnp.float32)


def _tc_mid(s, y, dinvb, b, w):
    return pl.pallas_call(
        _mid_kernel,
        grid=(GRID,),
        in_specs=[
            pl.BlockSpec((NC, BLK, D), lambda i: (0, i, 0)),
            pl.BlockSpec((BLK, D), lambda i: (i, 0)),
            pl.BlockSpec((BLK, D), lambda i: (i, 0)),
            pl.BlockSpec((1, D), lambda i: (0, 0)),
            pl.BlockSpec((D, D), lambda i: (0, 0)),
        ],
        out_specs=pl.BlockSpec((BLK, D), lambda i: (i, 0)),
        out_shape=jax.ShapeDtypeStruct((N, D), jnp.float32),
    )(s, y, dinvb, b, w)


def _fin_kernel(s_ref, y_ref, dinv_ref, b_ref, wfc_ref, bfc_ref, o_ref):
    t = dinv_ref[...] * (s_ref[0] + s_ref[1] + y_ref[...]) + b_ref[...]
    h = jnp.maximum(t, 0.0)
    logits = jnp.dot(h, wfc_ref[...],
                     preferred_element_type=jnp.float32) + bfc_ref[...]
    m = jnp.max(logits, axis=1, keepdims=True)
    lse = m + jnp.log(jnp.sum(jnp.exp(logits - m), axis=1, keepdims=True))
    o_ref[...] = logits - lse


def _tc_fin(s, y, dinvb, b, wfc_p, bfc_p):
    return pl.pallas_call(
        _fin_kernel,
        grid=(GRID,),
        in_specs=[
            pl.BlockSpec((NC, BLK, D), lambda i: (0, i, 0)),
            pl.BlockSpec((BLK, D), lambda i: (i, 0)),
            pl.BlockSpec((BLK, D), lambda i: (i, 0)),
            pl.BlockSpec((1, D), lambda i: (0, 0)),
            pl.BlockSpec((D, D), lambda i: (0, 0)),
            pl.BlockSpec((1, D), lambda i: (0, 0)),
        ],
        out_specs=pl.BlockSpec((BLK, D), lambda i: (i, 0)),
        out_shape=jax.ShapeDtypeStruct((N, D), jnp.float32),
    )(s, y, dinvb, b, wfc_p, bfc_p)


def kernel(x, edge_index, W0, b0, W1, b1, W2, b2, Wfc, bfc):
    src = edge_index[0]
    dst = edge_index[1]
    hist = _sc_deg(dst)
    y0, dinvb = _tc_prep(hist, x, W0)

    s0 = _sc_agg(y0, src, dst)
    y1 = _tc_mid(s0, y0, dinvb, b0.reshape(1, D), W1)

    s1 = _sc_agg(y1, src, dst)
    y2 = _tc_mid(s1, y1, dinvb, b1.reshape(1, D), W2)

    s2 = _sc_agg(y2, src, dst)
    wfc_p = jnp.zeros((D, D), jnp.float32).at[:, :OUT].set(Wfc)
    bfc_p = jnp.full((1, D), -1e30, jnp.float32).at[0, :OUT].set(bfc)
    logp = _tc_fin(s2, y2, dinvb, b2.reshape(1, D), wfc_p, bfc_p)
    return logp[:, :OUT]
